# baseline (speedup 1.0000x reference)
import functools

import jax
import jax.numpy as jnp
from jax import lax
from jax.experimental import pallas as pl
from jax.experimental.pallas import tpu as pltpu
from jax.experimental.pallas import tpu_sc as plsc

NPTS = 640000
NSEG = 10000
NSEGP = 10240
H = 64
NC, NS = 2, 16
NW = NC * NS
CHUNK = NPTS // NW
G = 8
NG = NPTS // G
GW = NG // NW
GCH = 50
FBR = 64
ECH = 400
NECH = CHUNK // ECH
ROWS_PER_S = NSEGP // NS

MB = 3200
NB = NPTS // MB

_NEG = float("-inf")


def _l1_body(p_ref, wp_ref, bp_ref, w1_ref, b1_ref, o_ref, gm_ref):
    a = jnp.dot(p_ref[...], wp_ref[...], preferred_element_type=jnp.float32)
    a = jnp.maximum(a + bp_ref[...], 0.0)
    x = jnp.dot(a, w1_ref[...], preferred_element_type=jnp.float32) + b1_ref[...]
    o_ref[...] = x
    gm_ref[...] = jnp.max(x.reshape(MB // G, G, H), axis=1)


def _layer1(pts, W_pos, b_pos, W1, b1):
    npr = pts.shape[0]
    return pl.pallas_call(
        _l1_body,
        grid=(npr // MB,),
        in_specs=[
            pl.BlockSpec((MB, 3), lambda i: (i, 0)),
            pl.BlockSpec((3, H), lambda i: (0, 0)),
            pl.BlockSpec((1, H), lambda i: (0, 0)),
            pl.BlockSpec((H, H), lambda i: (0, 0)),
            pl.BlockSpec((1, H), lambda i: (0, 0)),
        ],
        out_specs=[
            pl.BlockSpec((MB, H), lambda i: (i, 0)),
            pl.BlockSpec((MB // G, H), lambda i: (i, 0)),
        ],
        out_shape=[
            jax.ShapeDtypeStruct((npr, H), jnp.float32),
            jax.ShapeDtypeStruct((npr // G, H), jnp.float32),
        ],
    )(pts, W_pos, b_pos, W1, b1)


def _mid_body(h_ref, e_ref, w_ref, o_ref, gm_ref):
    x = jnp.maximum(h_ref[...], 0.0)
    x = jnp.dot(x, w_ref[...], preferred_element_type=jnp.float32) + e_ref[...]
    o_ref[...] = x
    gm_ref[...] = jnp.max(x.reshape(MB // G, G, H), axis=1)


def _mid(h, e, Wa):
    npr = h.shape[0]
    return pl.pallas_call(
        _mid_body,
        grid=(npr // MB,),
        in_specs=[
            pl.BlockSpec((MB, H), lambda i: (i, 0)),
            pl.BlockSpec((MB, H), lambda i: (i, 0)),
            pl.BlockSpec((H, H), lambda i: (0, 0)),
        ],
        out_specs=[
            pl.BlockSpec((MB, H), lambda i: (i, 0)),
            pl.BlockSpec((MB // G, H), lambda i: (i, 0)),
        ],
        out_shape=[
            jax.ShapeDtypeStruct((npr, H), jnp.float32),
            jax.ShapeDtypeStruct((npr // G, H), jnp.float32),
        ],
    )(h, e, Wa)


def _table4_body(T0_ref, T1_ref, T2_ref, T3_ref, w_ref, b_ref, o_ref):
    p = jnp.maximum(jnp.maximum(T0_ref[0], T0_ref[1]),
                    jnp.maximum(T1_ref[0], T1_ref[1]))
    q = jnp.maximum(jnp.maximum(T2_ref[0], T2_ref[1]),
                    jnp.maximum(T3_ref[0], T3_ref[1]))
    p = jnp.maximum(jnp.maximum(p, q), 0.0)
    o_ref[...] = jnp.dot(p, w_ref[...],
                         preferred_element_type=jnp.float32) + b_ref[...]


def _table4(Ts, Wb, b):
    return pl.pallas_call(
        _table4_body,
        grid=(1,),
        in_specs=[
            pl.BlockSpec((2, NSEGP, H), lambda i: (0, 0, 0)),
            pl.BlockSpec((2, NSEGP, H), lambda i: (0, 0, 0)),
            pl.BlockSpec((2, NSEGP, H), lambda i: (0, 0, 0)),
            pl.BlockSpec((2, NSEGP, H), lambda i: (0, 0, 0)),
            pl.BlockSpec((H, H), lambda i: (0, 0)),
            pl.BlockSpec((1, H), lambda i: (0, 0)),
        ],
        out_specs=pl.BlockSpec((NSEGP, H), lambda i: (0, 0)),
        out_shape=jax.ShapeDtypeStruct((NSEGP, H), jnp.float32),
    )(Ts[0], Ts[1], Ts[2], Ts[3], Wb, b)


def _table_body(TA_ref, TB_ref, w_ref, b_ref, o_ref):
    p = jnp.maximum(jnp.maximum(TA_ref[0], TA_ref[1]),
                    jnp.maximum(TB_ref[0], TB_ref[1]))
    p = jnp.maximum(p, 0.0)
    o_ref[...] = jnp.dot(p, w_ref[...],
                         preferred_element_type=jnp.float32) + b_ref[...]


def _table(TA, TB, Wb, b):
    return pl.pallas_call(
        _table_body,
        grid=(1,),
        in_specs=[
            pl.BlockSpec((2, NSEGP, H), lambda i: (0, 0, 0)),
            pl.BlockSpec((2, NSEGP, H), lambda i: (0, 0, 0)),
            pl.BlockSpec((H, H), lambda i: (0, 0)),
            pl.BlockSpec((1, H), lambda i: (0, 0)),
        ],
        out_specs=pl.BlockSpec((NSEGP, H), lambda i: (0, 0)),
        out_shape=jax.ShapeDtypeStruct((NSEGP, H), jnp.float32),
    )(TA, TB, Wb, b)


def _store_row(stg, a0, a1, a2, a3):
    stg[pl.ds(0, 16)] = a0
    stg[pl.ds(16, 16)] = a1
    stg[pl.ds(32, 16)] = a2
    stg[pl.ds(48, 16)] = a3


def _make_pool(NP, GCH=GCH, FBR=FBR):
    GWP = NP // G // NW
    NGCHP = GWP // GCH
    mesh = plsc.VectorSubcoreMesh(core_axis_name="c", subcore_axis_name="s",
                                  num_cores=NC, num_subcores=NS)

    @functools.partial(
        pl.kernel,
        out_type=jax.ShapeDtypeStruct((NC, NSEGP * H), jnp.float32),
        mesh=mesh,
        compiler_params=pltpu.CompilerParams(use_tc_tiling_on_sc=False),
        scratch_types=[
            pltpu.VMEM_SHARED((NSEGP * H,), jnp.float32),
            pltpu.VMEM_SHARED((2 * NS * H,), jnp.float32),
            pltpu.VMEM_SHARED((2 * NS * 16,), jnp.int32),
            pltpu.VMEM((GCH * H,), jnp.float32),
            pltpu.VMEM((GWP + 24,), jnp.int32),
            pltpu.VMEM((GWP + 24,), jnp.int32),
            pltpu.VMEM((GCH + 16,), jnp.int32),
            pltpu.VMEM((FBR, G * H), jnp.float32),
            pltpu.VMEM((FBR, 16), jnp.int32),
            pltpu.VMEM((64 * H,), jnp.float32),
            pltpu.VMEM((H,), jnp.float32),
            pltpu.VMEM((H,), jnp.float32),
            pltpu.VMEM((16,), jnp.int32),
            pltpu.VMEM((2 * NS * H,), jnp.float32),
            pltpu.VMEM((2 * NS * 16,), jnp.int32),
            pltpu.SemaphoreType.DMA,
            pltpu.SemaphoreType.DMA,
        ],
    )
    def k(h8_hbm, gm_hbm, igf_hbm, igl_hbm, idx16_hbm, out_hbm,
          t_sh, fl_sh, flid_sh, gmb, igfb, iglb, bndb, finebuf,
          fidxbuf, initbuf, accb, stg, istg, flbuf, idl, semf, semi):
        c = lax.axis_index("c")
        s = lax.axis_index("s")
        w = c * NS + s
        wg = w * GWP

        neg = jnp.full((16,), _NEG, jnp.float32)
        iota = lax.iota(jnp.int32, 16)

        def ifill(v, _):
            initbuf[pl.ds(v * 16, 16)] = neg
            return 0
        lax.fori_loop(0, 64 * H // 16, ifill, 0)

        def init_body(i, _):
            pltpu.sync_copy(
                initbuf,
                t_sh.at[pl.ds((s * ROWS_PER_S + i * 64) * H, 64 * H)])
            return 0
        lax.fori_loop(0, ROWS_PER_S // 64, init_body, 0)

        al = (wg // 8) * 8
        sh = wg - al
        pltpu.sync_copy(igf_hbm.at[pl.ds(al, GWP + 8)],
                        igfb.at[pl.ds(0, GWP + 8)])
        pltpu.sync_copy(igl_hbm.at[pl.ds(al, GWP + 8)],
                        iglb.at[pl.ds(0, GWP + 8)])
        for b in range((GCH + 16) // 16):
            bndb[pl.ds(b * 16, 16)] = jnp.zeros((16,), jnp.int32)
        plsc.subcore_barrier()

        def flush(nfl, cur):
            @pl.when(nfl == 0)
            def _():
                pltpu.sync_copy(accb, fl_sh.at[pl.ds((2 * s) * H, H)])
                istg[pl.ds(0, 16)] = jnp.full((16,), 1, jnp.int32) * cur
                pltpu.sync_copy(istg, flid_sh.at[pl.ds((2 * s) * 16, 16)])

            @pl.when(nfl > 0)
            def _():
                pltpu.sync_copy(accb, t_sh.at[pl.ds(cur * H, H)])

        cur0 = igfb[pl.ds(sh, 16)][0]
        for v in range(H // 16):
            accb[pl.ds(v * 16, 16)] = neg

        def chunk_body(ch, st):
            cur, nfl = st
            gbase = ch * GCH

            pltpu.sync_copy(gm_hbm.at[pl.ds((wg + gbase) * H, GCH * H)], gmb)

            def bscan(g, nb):
                sgF = igfb[pl.ds(sh + gbase + g, 16)][0]
                sgL = iglb[pl.ds(sh + gbase + g, 16)][0]
                imp = sgF != sgL

                @pl.when(imp)
                def _():
                    bndb[pl.ds(nb, 16)] = jnp.full(
                        (16,), 1, jnp.int32) * (wg + gbase + g)

                return nb + jnp.where(imp, 1, 0).astype(jnp.int32)
            nb = lax.fori_loop(0, GCH, bscan, jnp.int32(0))

            def gth(kk, _):
                ids = bndb.at[pl.ds(kk * 16, 16)]
                h1_ = pltpu.async_copy(h8_hbm.at[ids],
                                       finebuf.at[pl.ds(kk * 16, 16), :],
                                       semf)
                h2_ = pltpu.async_copy(idx16_hbm.at[ids],
                                       fidxbuf.at[pl.ds(kk * 16, 16), :],
                                       semi)
                h1_.wait()
                h2_.wait()
                return 0
            lax.fori_loop(0, (nb + 15) // 16, gth, 0)

            def gbody(g, st2):
                cur, nfl, fcur = st2
                sgF = igfb[pl.ds(sh + gbase + g, 16)][0]
                sgL = iglb[pl.ds(sh + gbase + g, 16)][0]
                impure = sgF != sgL

                def pure_fn(op):
                    cur, nfl, fcur = op
                    is_new = sgF != cur

                    @pl.when(is_new)
                    def _():
                        flush(nfl, cur)

                    for v in range(H // 16):
                        r = gmb[pl.ds(g * H + v * 16, 16)]
                        a = accb[pl.ds(v * 16, 16)]
                        accb[pl.ds(v * 16, 16)] = jnp.maximum(
                            jnp.where(is_new, neg, a), r)
                    return (sgF, jnp.where(is_new, nfl + 1, nfl), fcur)

                def fine_fn(op):
                    cur, nfl, fcur = op
                    idv = fidxbuf[fcur, pl.ds(0, 16)]
                    for kk in range(G):
                        sgk = idv[kk]
                        is_new = sgk != cur

                        @pl.when(is_new)
                        def _():
                            flush(nfl, cur)

                        for v in range(H // 16):
                            r = finebuf[fcur, pl.ds(kk * H + v * 16, 16)]
                            a = accb[pl.ds(v * 16, 16)]
                            accb[pl.ds(v * 16, 16)] = jnp.maximum(
                                jnp.where(is_new, neg, a), r)
                        nfl = jnp.where(is_new, nfl + 1, nfl)
                        cur = sgk
                    return (cur, nfl, fcur + 1)

                return lax.cond(impure, fine_fn, pure_fn, (cur, nfl, fcur))

            cur, nfl, _ = lax.fori_loop(
                0, GCH, gbody, (cur, nfl, jnp.int32(0)))
            return (cur, nfl)

        cur, nfl = lax.fori_loop(
            0, NGCHP, chunk_body, (cur0, jnp.int32(0)))

        @pl.when(nfl == 0)
        def _():
            pltpu.sync_copy(accb, fl_sh.at[pl.ds((2 * s) * H, H)])
            istg[pl.ds(0, 16)] = jnp.full((16,), 1, jnp.int32) * cur
            pltpu.sync_copy(istg, flid_sh.at[pl.ds((2 * s) * 16, 16)])
            istg[pl.ds(0, 16)] = jnp.full((16,), -1, jnp.int32)
            pltpu.sync_copy(istg, flid_sh.at[pl.ds((2 * s + 1) * 16, 16)])

        @pl.when(nfl > 0)
        def _():
            pltpu.sync_copy(accb, fl_sh.at[pl.ds((2 * s + 1) * H, H)])
            istg[pl.ds(0, 16)] = jnp.full((16,), 1, jnp.int32) * cur
            pltpu.sync_copy(istg, flid_sh.at[pl.ds((2 * s + 1) * 16, 16)])

        plsc.subcore_barrier()

        @pl.when(s == 0)
        def _():
            pltpu.sync_copy(fl_sh, flbuf)
            pltpu.sync_copy(flid_sh, idl)

            def cb(j, st):
                cur2, b0, b1, b2, b3 = st
                idj = idl[pl.ds(j * 16, 16)][0]
                n0 = flbuf[pl.ds(j * H, 16)]
                n1 = flbuf[pl.ds(j * H + 16, 16)]
                n2 = flbuf[pl.ds(j * H + 32, 16)]
                n3 = flbuf[pl.ds(j * H + 48, 16)]
                skip = idj < 0
                same = idj == cur2

                @pl.when(jnp.logical_and(~skip,
                                         jnp.logical_and(~same, cur2 >= 0)))
                def _():
                    _store_row(stg, b0, b1, b2, b3)
                    pltpu.sync_copy(stg, t_sh.at[pl.ds(cur2 * H, H)])

                ncur = jnp.where(skip, cur2, idj)
                fresh = jnp.logical_and(~skip, ~same)
                nb0 = jnp.where(skip, b0,
                                jnp.where(fresh, n0, jnp.maximum(b0, n0)))
                nb1 = jnp.where(skip, b1,
                                jnp.where(fresh, n1, jnp.maximum(b1, n1)))
                nb2 = jnp.where(skip, b2,
                                jnp.where(fresh, n2, jnp.maximum(b2, n2)))
                nb3 = jnp.where(skip, b3,
                                jnp.where(fresh, n3, jnp.maximum(b3, n3)))
                return (ncur, nb0, nb1, nb2, nb3)

            cur2, b0, b1, b2, b3 = lax.fori_loop(
                0, 2 * NS, cb, (jnp.int32(-1), neg, neg, neg, neg))

            @pl.when(cur2 >= 0)
            def _():
                _store_row(stg, b0, b1, b2, b3)
                pltpu.sync_copy(stg, t_sh.at[pl.ds(cur2 * H, H)])

        plsc.subcore_barrier()

        pltpu.sync_copy(
            t_sh.at[pl.ds(s * ROWS_PER_S * H, ROWS_PER_S * H)],
            out_hbm.at[c, pl.ds(s * ROWS_PER_S * H, ROWS_PER_S * H)])

    return k


def _make_expand(NP, ECH=ECH):
    CHP = NP // NW
    NECHP = CHP // ECH
    mesh = plsc.VectorSubcoreMesh(core_axis_name="c", subcore_axis_name="s",
                                  num_cores=NC, num_subcores=NS)

    @functools.partial(
        pl.kernel,
        out_type=jax.ShapeDtypeStruct((NP, H), jnp.float32),
        mesh=mesh,
        compiler_params=pltpu.CompilerParams(use_tc_tiling_on_sc=False),
        scratch_types=[
            pltpu.VMEM((ECH,), jnp.int32),
            pltpu.VMEM((ECH,), jnp.int32),
            pltpu.VMEM((ECH, H), jnp.float32),
            pltpu.VMEM((ECH, H), jnp.float32),
            pltpu.SemaphoreType.DMA,
            pltpu.SemaphoreType.DMA,
        ],
    )
    def k(t_hbm, idx_hbm, out_hbm, ib0, ib1, rb0, rb1, s0, s1):
        c = lax.axis_index("c")
        s = lax.axis_index("s")
        base = (c * NS + s) * CHP
        ibs = (ib0, ib1)
        rbs = (rb0, rb1)
        sems = (s0, s1)

        pltpu.sync_copy(idx_hbm.at[pl.ds(base, ECH)], ib0)
        gh = pltpu.async_copy(t_hbm.at[ib0], rb0, s0)
        for i in range(NECHP):
            p = i % 2
            q = (i + 1) % 2
            if i + 1 < NECHP:
                pltpu.sync_copy(
                    idx_hbm.at[pl.ds(base + (i + 1) * ECH, ECH)], ibs[q])
                gh_next = pltpu.async_copy(t_hbm.at[ibs[q]], rbs[q], sems[q])
            gh.wait()
            pltpu.sync_copy(rbs[p], out_hbm.at[pl.ds(base + i * ECH, ECH), :])
            if i + 1 < NECHP:
                gh = gh_next

    return k


def kernel(pts, idx, n_idx, W_pos, b_pos, W1, b1, W2, b2, W3, b3, W4, b4,
           W_out, b_out):
    idx = idx.astype(jnp.int32)
    bp = b_pos.reshape(1, H)
    b1r = b1.reshape(1, H)
    b2r = b2.reshape(1, H)
    b3r = b3.reshape(1, H)
    b4r = b4.reshape(1, H)
    bor = b_out.reshape(1, H)

    NSP = 4
    NPH = NPTS // NSP
    NGH = NPH // G
    pool_h = _make_pool(NPH, GCH=25, FBR=32)
    exp_h = _make_expand(NPH, ECH=200)

    parts = []
    for q in range(NSP):
        idxh = idx[q * NPH:(q + 1) * NPH]
        igfh = idxh[0::G]
        iglh = idxh[G - 1::G]
        idx16h = jnp.pad(idxh.reshape(NGH, G), ((0, 0), (0, 16 - G)))
        parts.append((idxh, igfh, iglh, idx16h))

    def pool(hh, gg, hv):
        _, igfh, iglh, idx16h = hv
        return pool_h(hh.reshape(NGH, G * H), gg.reshape(-1), igfh, iglh,
                      idx16h).reshape(NC, NSEGP, H)

    def expand(tt, hv):
        return exp_h(tt, hv[0])

    hs = []
    for q in range(NSP):
        hs.append(_layer1(pts[q * NPH:(q + 1) * NPH], W_pos, bp, W1, b1r))

    for (Wk, bk) in ((W2, b2r), (W3, b3r), (W4, b4r)):
        Ts = [pool(hq, gq, parts[q]) for q, (hq, gq) in enumerate(hs)]
        t = _table4(Ts, Wk[H:], bk)
        hs = [_mid(hs[q][0], expand(t, parts[q]), Wk[:H])
              for q in range(NSP)]

    Ts = [pool(hq, gq, parts[q]) for q, (hq, gq) in enumerate(hs)]
    out = _table4(Ts, W_out, bor)
    return out[:NSEG]


# speedup vs baseline: 1.1655x; 1.1655x over previous
import functools

import jax
import jax.numpy as jnp
from jax import lax
from jax.experimental import pallas as pl
from jax.experimental.pallas import tpu as pltpu
from jax.experimental.pallas import tpu_sc as plsc

NPTS = 640000
NSEG = 10000
NSEGP = 10240
H = 64
NC, NS = 2, 16
NW = NC * NS
CHUNK = NPTS // NW
G = 8
NG = NPTS // G
GW = NG // NW
GCH = 50
FBR = 64
ECH = 400
NECH = CHUNK // ECH
ROWS_PER_S = NSEGP // NS

MB = 3200
NB = NPTS // MB

_NEG = float("-inf")


def _l1_body(p_ref, wp_ref, bp_ref, w1_ref, b1_ref, o_ref, gm_ref):
    a = jnp.dot(p_ref[...], wp_ref[...], preferred_element_type=jnp.float32)
    a = jnp.maximum(a + bp_ref[...], 0.0)
    x = jnp.dot(a, w1_ref[...], preferred_element_type=jnp.float32) + b1_ref[...]
    o_ref[...] = x
    gm_ref[...] = jnp.max(x.reshape(MB // G, G, H), axis=1)


def _layer1(pts, W_pos, b_pos, W1, b1):
    npr = pts.shape[0]
    return pl.pallas_call(
        _l1_body,
        grid=(npr // MB,),
        in_specs=[
            pl.BlockSpec((MB, 3), lambda i: (i, 0)),
            pl.BlockSpec((3, H), lambda i: (0, 0)),
            pl.BlockSpec((1, H), lambda i: (0, 0)),
            pl.BlockSpec((H, H), lambda i: (0, 0)),
            pl.BlockSpec((1, H), lambda i: (0, 0)),
        ],
        out_specs=[
            pl.BlockSpec((MB, H), lambda i: (i, 0)),
            pl.BlockSpec((MB // G, H), lambda i: (i, 0)),
        ],
        out_shape=[
            jax.ShapeDtypeStruct((npr, H), jnp.float32),
            jax.ShapeDtypeStruct((npr // G, H), jnp.float32),
        ],
    )(pts, W_pos, b_pos, W1, b1)


def _mid_body(h_ref, e_ref, w_ref, o_ref, gm_ref):
    x = jnp.maximum(h_ref[...], 0.0)
    x = jnp.dot(x, w_ref[...], preferred_element_type=jnp.float32) + e_ref[...]
    o_ref[...] = x
    gm_ref[...] = jnp.max(x.reshape(MB // G, G, H), axis=1)


def _mid(h, e, Wa):
    npr = h.shape[0]
    return pl.pallas_call(
        _mid_body,
        grid=(npr // MB,),
        in_specs=[
            pl.BlockSpec((MB, H), lambda i: (i, 0)),
            pl.BlockSpec((MB, H), lambda i: (i, 0)),
            pl.BlockSpec((H, H), lambda i: (0, 0)),
        ],
        out_specs=[
            pl.BlockSpec((MB, H), lambda i: (i, 0)),
            pl.BlockSpec((MB // G, H), lambda i: (i, 0)),
        ],
        out_shape=[
            jax.ShapeDtypeStruct((npr, H), jnp.float32),
            jax.ShapeDtypeStruct((npr // G, H), jnp.float32),
        ],
    )(h, e, Wa)


def _table4_body(T0_ref, T1_ref, T2_ref, T3_ref, w_ref, b_ref, o_ref):
    p = jnp.maximum(jnp.maximum(T0_ref[0], T0_ref[1]),
                    jnp.maximum(T1_ref[0], T1_ref[1]))
    q = jnp.maximum(jnp.maximum(T2_ref[0], T2_ref[1]),
                    jnp.maximum(T3_ref[0], T3_ref[1]))
    p = jnp.maximum(jnp.maximum(p, q), 0.0)
    o_ref[...] = jnp.dot(p, w_ref[...],
                         preferred_element_type=jnp.float32) + b_ref[...]


def _table5_body(T0_ref, T1_ref, T2_ref, T3_ref, T4_ref, w_ref, b_ref,
                 o_ref):
    p = jnp.maximum(jnp.maximum(T0_ref[0], T0_ref[1]),
                    jnp.maximum(T1_ref[0], T1_ref[1]))
    q = jnp.maximum(jnp.maximum(T2_ref[0], T2_ref[1]),
                    jnp.maximum(T3_ref[0], T3_ref[1]))
    p = jnp.maximum(p, jnp.maximum(T4_ref[0], T4_ref[1]))
    p = jnp.maximum(jnp.maximum(p, q), 0.0)
    o_ref[...] = jnp.dot(p, w_ref[...],
                         preferred_element_type=jnp.float32) + b_ref[...]


def _table5(Ts, Wb, b):
    return pl.pallas_call(
        _table5_body,
        grid=(1,),
        in_specs=[pl.BlockSpec((2, NSEGP, H), lambda i: (0, 0, 0))
                  for _ in range(5)] + [
            pl.BlockSpec((H, H), lambda i: (0, 0)),
            pl.BlockSpec((1, H), lambda i: (0, 0)),
        ],
        out_specs=pl.BlockSpec((NSEGP, H), lambda i: (0, 0)),
        out_shape=jax.ShapeDtypeStruct((NSEGP, H), jnp.float32),
    )(Ts[0], Ts[1], Ts[2], Ts[3], Ts[4], Wb, b)


def _table_body(TA_ref, TB_ref, w_ref, b_ref, o_ref):
    p = jnp.maximum(jnp.maximum(TA_ref[0], TA_ref[1]),
                    jnp.maximum(TB_ref[0], TB_ref[1]))
    p = jnp.maximum(p, 0.0)
    o_ref[...] = jnp.dot(p, w_ref[...],
                         preferred_element_type=jnp.float32) + b_ref[...]


def _table(TA, TB, Wb, b):
    return pl.pallas_call(
        _table_body,
        grid=(1,),
        in_specs=[
            pl.BlockSpec((2, NSEGP, H), lambda i: (0, 0, 0)),
            pl.BlockSpec((2, NSEGP, H), lambda i: (0, 0, 0)),
            pl.BlockSpec((H, H), lambda i: (0, 0)),
            pl.BlockSpec((1, H), lambda i: (0, 0)),
        ],
        out_specs=pl.BlockSpec((NSEGP, H), lambda i: (0, 0)),
        out_shape=jax.ShapeDtypeStruct((NSEGP, H), jnp.float32),
    )(TA, TB, Wb, b)


def _store_row(stg, a0, a1, a2, a3):
    stg[pl.ds(0, 16)] = a0
    stg[pl.ds(16, 16)] = a1
    stg[pl.ds(32, 16)] = a2
    stg[pl.ds(48, 16)] = a3


def _make_pool(NP, GCH=GCH, FBR=FBR):
    GWP = NP // G // NW
    NGCHP = GWP // GCH
    mesh = plsc.VectorSubcoreMesh(core_axis_name="c", subcore_axis_name="s",
                                  num_cores=NC, num_subcores=NS)

    @functools.partial(
        pl.kernel,
        out_type=jax.ShapeDtypeStruct((NC, NSEGP * H), jnp.float32),
        mesh=mesh,
        compiler_params=pltpu.CompilerParams(use_tc_tiling_on_sc=False),
        scratch_types=[
            pltpu.VMEM_SHARED((NSEGP * H,), jnp.float32),
            pltpu.VMEM_SHARED((2 * NS * H,), jnp.float32),
            pltpu.VMEM_SHARED((2 * NS * 16,), jnp.int32),
            pltpu.VMEM((GCH * H,), jnp.float32),
            pltpu.VMEM((GWP + 24,), jnp.int32),
            pltpu.VMEM((GWP + 24,), jnp.int32),
            pltpu.VMEM((GCH + 16,), jnp.int32),
            pltpu.VMEM((FBR, G * H), jnp.float32),
            pltpu.VMEM((FBR, 16), jnp.int32),
            pltpu.VMEM((64 * H,), jnp.float32),
            pltpu.VMEM((H,), jnp.float32),
            pltpu.VMEM((H,), jnp.float32),
            pltpu.VMEM((16,), jnp.int32),
            pltpu.VMEM((2 * NS * H,), jnp.float32),
            pltpu.VMEM((2 * NS * 16,), jnp.int32),
            pltpu.SemaphoreType.DMA,
            pltpu.SemaphoreType.DMA,
        ],
    )
    def k(h8_hbm, gm_hbm, igf_hbm, igl_hbm, idx16_hbm, out_hbm,
          t_sh, fl_sh, flid_sh, gmb, igfb, iglb, bndb, finebuf,
          fidxbuf, initbuf, accb, stg, istg, flbuf, idl, semf, semi):
        c = lax.axis_index("c")
        s = lax.axis_index("s")
        w = c * NS + s
        wg = w * GWP

        neg = jnp.full((16,), _NEG, jnp.float32)
        iota = lax.iota(jnp.int32, 16)

        def ifill(v, _):
            initbuf[pl.ds(v * 16, 16)] = neg
            return 0
        lax.fori_loop(0, 64 * H // 16, ifill, 0)

        def init_body(i, _):
            pltpu.sync_copy(
                initbuf,
                t_sh.at[pl.ds((s * ROWS_PER_S + i * 64) * H, 64 * H)])
            return 0
        lax.fori_loop(0, ROWS_PER_S // 64, init_body, 0)

        al = (wg // 8) * 8
        sh = wg - al
        pltpu.sync_copy(igf_hbm.at[pl.ds(al, GWP + 8)],
                        igfb.at[pl.ds(0, GWP + 8)])
        pltpu.sync_copy(igl_hbm.at[pl.ds(al, GWP + 8)],
                        iglb.at[pl.ds(0, GWP + 8)])
        for b in range((GCH + 16) // 16):
            bndb[pl.ds(b * 16, 16)] = jnp.zeros((16,), jnp.int32)
        plsc.subcore_barrier()

        def flush(nfl, cur):
            @pl.when(nfl == 0)
            def _():
                pltpu.sync_copy(accb, fl_sh.at[pl.ds((2 * s) * H, H)])
                istg[pl.ds(0, 16)] = jnp.full((16,), 1, jnp.int32) * cur
                pltpu.sync_copy(istg, flid_sh.at[pl.ds((2 * s) * 16, 16)])

            @pl.when(nfl > 0)
            def _():
                pltpu.sync_copy(accb, t_sh.at[pl.ds(cur * H, H)])

        cur0 = igfb[pl.ds(sh, 16)][0]
        for v in range(H // 16):
            accb[pl.ds(v * 16, 16)] = neg

        def chunk_body(ch, st):
            cur, nfl = st
            gbase = ch * GCH

            pltpu.sync_copy(gm_hbm.at[pl.ds((wg + gbase) * H, GCH * H)], gmb)

            def bscan(g, nb):
                sgF = igfb[pl.ds(sh + gbase + g, 16)][0]
                sgL = iglb[pl.ds(sh + gbase + g, 16)][0]
                imp = sgF != sgL

                @pl.when(imp)
                def _():
                    bndb[pl.ds(nb, 16)] = jnp.full(
                        (16,), 1, jnp.int32) * (wg + gbase + g)

                return nb + jnp.where(imp, 1, 0).astype(jnp.int32)
            nb = lax.fori_loop(0, GCH, bscan, jnp.int32(0))

            def gth(kk, _):
                ids = bndb.at[pl.ds(kk * 16, 16)]
                h1_ = pltpu.async_copy(h8_hbm.at[ids],
                                       finebuf.at[pl.ds(kk * 16, 16), :],
                                       semf)
                h2_ = pltpu.async_copy(idx16_hbm.at[ids],
                                       fidxbuf.at[pl.ds(kk * 16, 16), :],
                                       semi)
                h1_.wait()
                h2_.wait()
                return 0
            lax.fori_loop(0, (nb + 15) // 16, gth, 0)

            def gbody(g, st2):
                cur, nfl, fcur = st2
                sgF = igfb[pl.ds(sh + gbase + g, 16)][0]
                sgL = iglb[pl.ds(sh + gbase + g, 16)][0]
                impure = sgF != sgL

                def pure_fn(op):
                    cur, nfl, fcur = op
                    is_new = sgF != cur

                    @pl.when(is_new)
                    def _():
                        flush(nfl, cur)

                    for v in range(H // 16):
                        r = gmb[pl.ds(g * H + v * 16, 16)]
                        a = accb[pl.ds(v * 16, 16)]
                        accb[pl.ds(v * 16, 16)] = jnp.maximum(
                            jnp.where(is_new, neg, a), r)
                    return (sgF, jnp.where(is_new, nfl + 1, nfl), fcur)

                def fine_fn(op):
                    cur, nfl, fcur = op
                    idv = fidxbuf[fcur, pl.ds(0, 16)]
                    for kk in range(G):
                        sgk = idv[kk]
                        is_new = sgk != cur

                        @pl.when(is_new)
                        def _():
                            flush(nfl, cur)

                        for v in range(H // 16):
                            r = finebuf[fcur, pl.ds(kk * H + v * 16, 16)]
                            a = accb[pl.ds(v * 16, 16)]
                            accb[pl.ds(v * 16, 16)] = jnp.maximum(
                                jnp.where(is_new, neg, a), r)
                        nfl = jnp.where(is_new, nfl + 1, nfl)
                        cur = sgk
                    return (cur, nfl, fcur + 1)

                return lax.cond(impure, fine_fn, pure_fn, (cur, nfl, fcur))

            cur, nfl, _ = lax.fori_loop(
                0, GCH, gbody, (cur, nfl, jnp.int32(0)))
            return (cur, nfl)

        cur, nfl = lax.fori_loop(
            0, NGCHP, chunk_body, (cur0, jnp.int32(0)))

        @pl.when(nfl == 0)
        def _():
            pltpu.sync_copy(accb, fl_sh.at[pl.ds((2 * s) * H, H)])
            istg[pl.ds(0, 16)] = jnp.full((16,), 1, jnp.int32) * cur
            pltpu.sync_copy(istg, flid_sh.at[pl.ds((2 * s) * 16, 16)])
            istg[pl.ds(0, 16)] = jnp.full((16,), -1, jnp.int32)
            pltpu.sync_copy(istg, flid_sh.at[pl.ds((2 * s + 1) * 16, 16)])

        @pl.when(nfl > 0)
        def _():
            pltpu.sync_copy(accb, fl_sh.at[pl.ds((2 * s + 1) * H, H)])
            istg[pl.ds(0, 16)] = jnp.full((16,), 1, jnp.int32) * cur
            pltpu.sync_copy(istg, flid_sh.at[pl.ds((2 * s + 1) * 16, 16)])

        plsc.subcore_barrier()

        @pl.when(s == 0)
        def _():
            pltpu.sync_copy(fl_sh, flbuf)
            pltpu.sync_copy(flid_sh, idl)

            def cb(j, st):
                cur2, b0, b1, b2, b3 = st
                idj = idl[pl.ds(j * 16, 16)][0]
                n0 = flbuf[pl.ds(j * H, 16)]
                n1 = flbuf[pl.ds(j * H + 16, 16)]
                n2 = flbuf[pl.ds(j * H + 32, 16)]
                n3 = flbuf[pl.ds(j * H + 48, 16)]
                skip = idj < 0
                same = idj == cur2

                @pl.when(jnp.logical_and(~skip,
                                         jnp.logical_and(~same, cur2 >= 0)))
                def _():
                    _store_row(stg, b0, b1, b2, b3)
                    pltpu.sync_copy(stg, t_sh.at[pl.ds(cur2 * H, H)])

                ncur = jnp.where(skip, cur2, idj)
                fresh = jnp.logical_and(~skip, ~same)
                nb0 = jnp.where(skip, b0,
                                jnp.where(fresh, n0, jnp.maximum(b0, n0)))
                nb1 = jnp.where(skip, b1,
                                jnp.where(fresh, n1, jnp.maximum(b1, n1)))
                nb2 = jnp.where(skip, b2,
                                jnp.where(fresh, n2, jnp.maximum(b2, n2)))
                nb3 = jnp.where(skip, b3,
                                jnp.where(fresh, n3, jnp.maximum(b3, n3)))
                return (ncur, nb0, nb1, nb2, nb3)

            cur2, b0, b1, b2, b3 = lax.fori_loop(
                0, 2 * NS, cb, (jnp.int32(-1), neg, neg, neg, neg))

            @pl.when(cur2 >= 0)
            def _():
                _store_row(stg, b0, b1, b2, b3)
                pltpu.sync_copy(stg, t_sh.at[pl.ds(cur2 * H, H)])

        plsc.subcore_barrier()

        pltpu.sync_copy(
            t_sh.at[pl.ds(s * ROWS_PER_S * H, ROWS_PER_S * H)],
            out_hbm.at[c, pl.ds(s * ROWS_PER_S * H, ROWS_PER_S * H)])

    return k


def _make_expand(NP, ECH=ECH):
    CHP = NP // NW
    NECHP = CHP // ECH
    mesh = plsc.VectorSubcoreMesh(core_axis_name="c", subcore_axis_name="s",
                                  num_cores=NC, num_subcores=NS)

    @functools.partial(
        pl.kernel,
        out_type=jax.ShapeDtypeStruct((NP, H), jnp.float32),
        mesh=mesh,
        compiler_params=pltpu.CompilerParams(use_tc_tiling_on_sc=False),
        scratch_types=[
            pltpu.VMEM((ECH,), jnp.int32),
            pltpu.VMEM((ECH,), jnp.int32),
            pltpu.VMEM((ECH, H), jnp.float32),
            pltpu.VMEM((ECH, H), jnp.float32),
            pltpu.SemaphoreType.DMA,
            pltpu.SemaphoreType.DMA,
        ],
    )
    def k(t_hbm, idx_hbm, out_hbm, ib0, ib1, rb0, rb1, s0, s1):
        c = lax.axis_index("c")
        s = lax.axis_index("s")
        base = (c * NS + s) * CHP
        ibs = (ib0, ib1)
        rbs = (rb0, rb1)
        sems = (s0, s1)

        pltpu.sync_copy(idx_hbm.at[pl.ds(base, ECH)], ib0)
        gh = pltpu.async_copy(t_hbm.at[ib0], rb0, s0)
        for i in range(NECHP):
            p = i % 2
            q = (i + 1) % 2
            if i + 1 < NECHP:
                pltpu.sync_copy(
                    idx_hbm.at[pl.ds(base + (i + 1) * ECH, ECH)], ibs[q])
                gh_next = pltpu.async_copy(t_hbm.at[ibs[q]], rbs[q], sems[q])
            gh.wait()
            pltpu.sync_copy(rbs[p], out_hbm.at[pl.ds(base + i * ECH, ECH), :])
            if i + 1 < NECHP:
                gh = gh_next

    return k


def kernel(pts, idx, n_idx, W_pos, b_pos, W1, b1, W2, b2, W3, b3, W4, b4,
           W_out, b_out):
    idx = idx.astype(jnp.int32)
    bp = b_pos.reshape(1, H)
    b1r = b1.reshape(1, H)
    b2r = b2.reshape(1, H)
    b3r = b3.reshape(1, H)
    b4r = b4.reshape(1, H)
    bor = b_out.reshape(1, H)

    NSP = 5
    NPH = NPTS // NSP
    NGH = NPH // G
    pool_h = _make_pool(NPH, GCH=100, FBR=112)
    exp_h = _make_expand(NPH, ECH=400)

    parts = []
    for q in range(NSP):
        idxh = idx[q * NPH:(q + 1) * NPH]
        igfh = idxh[0::G]
        iglh = idxh[G - 1::G]
        idx16h = jnp.pad(idxh.reshape(NGH, G), ((0, 0), (0, 16 - G)))
        parts.append((idxh, igfh, iglh, idx16h))

    def pool(hh, gg, hv):
        _, igfh, iglh, idx16h = hv
        return pool_h(hh.reshape(NGH, G * H), gg.reshape(-1), igfh, iglh,
                      idx16h).reshape(NC, NSEGP, H)

    def expand(tt, hv):
        return exp_h(tt, hv[0])

    hs = []
    for q in range(NSP):
        hs.append(_layer1(pts[q * NPH:(q + 1) * NPH], W_pos, bp, W1, b1r))

    for (Wk, bk) in ((W2, b2r), (W3, b3r), (W4, b4r)):
        Ts = [pool(hq, gq, parts[q]) for q, (hq, gq) in enumerate(hs)]
        t = _table5(Ts, Wk[H:], bk)
        hs = [_mid(hs[q][0], expand(t, parts[q]), Wk[:H])
              for q in range(NSP)]

    Ts = [pool(hq, gq, parts[q]) for q, (hq, gq) in enumerate(hs)]
    out = _table5(Ts, W_out, bor)
    return out[:NSEG]


# speedup vs baseline: 1.1857x; 1.0174x over previous
import functools

import jax
import jax.numpy as jnp
from jax import lax
from jax.experimental import pallas as pl
from jax.experimental.pallas import tpu as pltpu
from jax.experimental.pallas import tpu_sc as plsc

NPTS = 640000
NSEG = 10000
NSEGP = 10240
H = 64
NC, NS = 2, 16
NW = NC * NS
CHUNK = NPTS // NW
G = 8
NG = NPTS // G
GW = NG // NW
GCH = 50
FBR = 64
ECH = 400
NECH = CHUNK // ECH
ROWS_PER_S = NSEGP // NS

MB = 3200
NB = NPTS // MB

_NEG = float("-inf")


def _l1_body(p_ref, wp_ref, bp_ref, w1_ref, b1_ref, o_ref, gm_ref):
    a = jnp.dot(p_ref[...], wp_ref[...], preferred_element_type=jnp.float32)
    a = jnp.maximum(a + bp_ref[...], 0.0)
    x = jnp.dot(a, w1_ref[...], preferred_element_type=jnp.float32) + b1_ref[...]
    o_ref[...] = x
    gm_ref[...] = jnp.max(x.reshape(MB // G, G, H), axis=1)


def _layer1(pts, W_pos, b_pos, W1, b1):
    npr = pts.shape[0]
    return pl.pallas_call(
        _l1_body,
        grid=(npr // MB,),
        in_specs=[
            pl.BlockSpec((MB, 3), lambda i: (i, 0)),
            pl.BlockSpec((3, H), lambda i: (0, 0)),
            pl.BlockSpec((1, H), lambda i: (0, 0)),
            pl.BlockSpec((H, H), lambda i: (0, 0)),
            pl.BlockSpec((1, H), lambda i: (0, 0)),
        ],
        out_specs=[
            pl.BlockSpec((MB, H), lambda i: (i, 0)),
            pl.BlockSpec((MB // G, H), lambda i: (i, 0)),
        ],
        out_shape=[
            jax.ShapeDtypeStruct((npr, H), jnp.float32),
            jax.ShapeDtypeStruct((npr // G, H), jnp.float32),
        ],
    )(pts, W_pos, b_pos, W1, b1)


def _mid_body(h_ref, e_ref, w_ref, o_ref, gm_ref):
    x = jnp.maximum(h_ref[...], 0.0)
    x = jnp.dot(x, w_ref[...], preferred_element_type=jnp.float32) + e_ref[...]
    o_ref[...] = x
    gm_ref[...] = jnp.max(x.reshape(MB // G, G, H), axis=1)


def _mid(h, e, Wa):
    npr = h.shape[0]
    return pl.pallas_call(
        _mid_body,
        grid=(npr // MB,),
        in_specs=[
            pl.BlockSpec((MB, H), lambda i: (i, 0)),
            pl.BlockSpec((MB, H), lambda i: (i, 0)),
            pl.BlockSpec((H, H), lambda i: (0, 0)),
        ],
        out_specs=[
            pl.BlockSpec((MB, H), lambda i: (i, 0)),
            pl.BlockSpec((MB // G, H), lambda i: (i, 0)),
        ],
        out_shape=[
            jax.ShapeDtypeStruct((npr, H), jnp.float32),
            jax.ShapeDtypeStruct((npr // G, H), jnp.float32),
        ],
    )(h, e, Wa)


def _table4_body(T0_ref, T1_ref, T2_ref, T3_ref, w_ref, b_ref, o_ref):
    p = jnp.maximum(jnp.maximum(T0_ref[0], T0_ref[1]),
                    jnp.maximum(T1_ref[0], T1_ref[1]))
    q = jnp.maximum(jnp.maximum(T2_ref[0], T2_ref[1]),
                    jnp.maximum(T3_ref[0], T3_ref[1]))
    p = jnp.maximum(jnp.maximum(p, q), 0.0)
    o_ref[...] = jnp.dot(p, w_ref[...],
                         preferred_element_type=jnp.float32) + b_ref[...]


def _table5_body(T0_ref, T1_ref, T2_ref, T3_ref, T4_ref, w_ref, b_ref,
                 o_ref):
    p = jnp.maximum(jnp.maximum(T0_ref[0], T0_ref[1]),
                    jnp.maximum(T1_ref[0], T1_ref[1]))
    q = jnp.maximum(jnp.maximum(T2_ref[0], T2_ref[1]),
                    jnp.maximum(T3_ref[0], T3_ref[1]))
    p = jnp.maximum(p, jnp.maximum(T4_ref[0], T4_ref[1]))
    p = jnp.maximum(jnp.maximum(p, q), 0.0)
    o_ref[...] = jnp.dot(p, w_ref[...],
                         preferred_element_type=jnp.float32) + b_ref[...]


def _table5(Ts, Wb, b):
    return pl.pallas_call(
        _table5_body,
        grid=(1,),
        in_specs=[pl.BlockSpec((2, NSEGP, H), lambda i: (0, 0, 0))
                  for _ in range(5)] + [
            pl.BlockSpec((H, H), lambda i: (0, 0)),
            pl.BlockSpec((1, H), lambda i: (0, 0)),
        ],
        out_specs=pl.BlockSpec((NSEGP, H), lambda i: (0, 0)),
        out_shape=jax.ShapeDtypeStruct((NSEGP, H), jnp.float32),
    )(Ts[0], Ts[1], Ts[2], Ts[3], Ts[4], Wb, b)


def _table_body(TA_ref, TB_ref, w_ref, b_ref, o_ref):
    p = jnp.maximum(jnp.maximum(TA_ref[0], TA_ref[1]),
                    jnp.maximum(TB_ref[0], TB_ref[1]))
    p = jnp.maximum(p, 0.0)
    o_ref[...] = jnp.dot(p, w_ref[...],
                         preferred_element_type=jnp.float32) + b_ref[...]


def _table(TA, TB, Wb, b):
    return pl.pallas_call(
        _table_body,
        grid=(1,),
        in_specs=[
            pl.BlockSpec((2, NSEGP, H), lambda i: (0, 0, 0)),
            pl.BlockSpec((2, NSEGP, H), lambda i: (0, 0, 0)),
            pl.BlockSpec((H, H), lambda i: (0, 0)),
            pl.BlockSpec((1, H), lambda i: (0, 0)),
        ],
        out_specs=pl.BlockSpec((NSEGP, H), lambda i: (0, 0)),
        out_shape=jax.ShapeDtypeStruct((NSEGP, H), jnp.float32),
    )(TA, TB, Wb, b)


def _store_row(stg, a0, a1, a2, a3):
    stg[pl.ds(0, 16)] = a0
    stg[pl.ds(16, 16)] = a1
    stg[pl.ds(32, 16)] = a2
    stg[pl.ds(48, 16)] = a3


def _make_pool(NP, GCH=GCH, FBR=FBR):
    GWP = NP // G // NW
    NGCHP = GWP // GCH
    mesh = plsc.VectorSubcoreMesh(core_axis_name="c", subcore_axis_name="s",
                                  num_cores=NC, num_subcores=NS)

    @functools.partial(
        pl.kernel,
        out_type=jax.ShapeDtypeStruct((NC, NSEGP * H), jnp.float32),
        mesh=mesh,
        compiler_params=pltpu.CompilerParams(use_tc_tiling_on_sc=False),
        scratch_types=[
            pltpu.VMEM_SHARED((NSEGP * H,), jnp.float32),
            pltpu.VMEM_SHARED((2 * NS * H,), jnp.float32),
            pltpu.VMEM_SHARED((2 * NS * 16,), jnp.int32),
            pltpu.VMEM((GCH * H,), jnp.float32),
            pltpu.VMEM((GWP + 24,), jnp.int32),
            pltpu.VMEM((GWP + 24,), jnp.int32),
            pltpu.VMEM((GCH + 16,), jnp.int32),
            pltpu.VMEM((FBR, G * H), jnp.float32),
            pltpu.VMEM((FBR, 16), jnp.int32),
            pltpu.VMEM((64 * H,), jnp.float32),
            pltpu.VMEM((H,), jnp.float32),
            pltpu.VMEM((H,), jnp.float32),
            pltpu.VMEM((16,), jnp.int32),
            pltpu.VMEM((2 * NS * H,), jnp.float32),
            pltpu.VMEM((2 * NS * 16,), jnp.int32),
            pltpu.SemaphoreType.DMA,
            pltpu.SemaphoreType.DMA,
        ],
    )
    def k(h8_hbm, gm_hbm, igf_hbm, igl_hbm, idx16_hbm, out_hbm,
          t_sh, fl_sh, flid_sh, gmb, igfb, iglb, bndb, finebuf,
          fidxbuf, initbuf, accb, stg, istg, flbuf, idl, semf, semi):
        c = lax.axis_index("c")
        s = lax.axis_index("s")
        w = c * NS + s
        wg = w * GWP

        neg = jnp.full((16,), _NEG, jnp.float32)
        iota = lax.iota(jnp.int32, 16)

        def ifill(v, _):
            initbuf[pl.ds(v * 16, 16)] = neg
            return 0
        lax.fori_loop(0, 64 * H // 16, ifill, 0)

        def init_body(i, _):
            pltpu.sync_copy(
                initbuf,
                t_sh.at[pl.ds((s * ROWS_PER_S + i * 64) * H, 64 * H)])
            return 0
        lax.fori_loop(0, ROWS_PER_S // 64, init_body, 0)

        al = (wg // 8) * 8
        sh = wg - al
        pltpu.sync_copy(igf_hbm.at[pl.ds(al, GWP + 8)],
                        igfb.at[pl.ds(0, GWP + 8)])
        pltpu.sync_copy(igl_hbm.at[pl.ds(al, GWP + 8)],
                        iglb.at[pl.ds(0, GWP + 8)])
        for b in range((GCH + 16) // 16):
            bndb[pl.ds(b * 16, 16)] = jnp.zeros((16,), jnp.int32)
        plsc.subcore_barrier()

        def flush(nfl, cur):
            @pl.when(nfl == 0)
            def _():
                pltpu.sync_copy(accb, fl_sh.at[pl.ds((2 * s) * H, H)])
                istg[pl.ds(0, 16)] = jnp.full((16,), 1, jnp.int32) * cur
                pltpu.sync_copy(istg, flid_sh.at[pl.ds((2 * s) * 16, 16)])

            @pl.when(nfl > 0)
            def _():
                pltpu.sync_copy(accb, t_sh.at[pl.ds(cur * H, H)])

        cur0 = igfb[pl.ds(sh, 16)][0]
        for v in range(H // 16):
            accb[pl.ds(v * 16, 16)] = neg

        def chunk_body(ch, st):
            cur, nfl = st
            gbase = ch * GCH

            pltpu.sync_copy(gm_hbm.at[pl.ds((wg + gbase) * H, GCH * H)], gmb)

            def bscan(g, nb):
                sgF = igfb[pl.ds(sh + gbase + g, 16)][0]
                sgL = iglb[pl.ds(sh + gbase + g, 16)][0]
                imp = sgF != sgL

                @pl.when(imp)
                def _():
                    bndb[pl.ds(nb, 16)] = jnp.full(
                        (16,), 1, jnp.int32) * (wg + gbase + g)

                return nb + jnp.where(imp, 1, 0).astype(jnp.int32)
            nb = lax.fori_loop(0, GCH, bscan, jnp.int32(0))

            def gth(kk, _):
                ids = bndb.at[pl.ds(kk * 16, 16)]
                h1_ = pltpu.async_copy(h8_hbm.at[ids],
                                       finebuf.at[pl.ds(kk * 16, 16), :],
                                       semf)
                h2_ = pltpu.async_copy(idx16_hbm.at[ids],
                                       fidxbuf.at[pl.ds(kk * 16, 16), :],
                                       semi)
                h1_.wait()
                h2_.wait()
                return 0
            lax.fori_loop(0, (nb + 15) // 16, gth, 0)

            def gbody(g, st2):
                cur, nfl, fcur = st2
                sgF = igfb[pl.ds(sh + gbase + g, 16)][0]
                sgL = iglb[pl.ds(sh + gbase + g, 16)][0]
                impure = sgF != sgL

                def pure_fn(op):
                    cur, nfl, fcur = op
                    is_new = sgF != cur

                    @pl.when(is_new)
                    def _():
                        flush(nfl, cur)

                    for v in range(H // 16):
                        r = gmb[pl.ds(g * H + v * 16, 16)]
                        a = accb[pl.ds(v * 16, 16)]
                        accb[pl.ds(v * 16, 16)] = jnp.maximum(
                            jnp.where(is_new, neg, a), r)
                    return (sgF, jnp.where(is_new, nfl + 1, nfl), fcur)

                def fine_fn(op):
                    cur, nfl, fcur = op
                    idv = fidxbuf[fcur, pl.ds(0, 16)]
                    for kk in range(G):
                        sgk = idv[kk]
                        is_new = sgk != cur

                        @pl.when(is_new)
                        def _():
                            flush(nfl, cur)

                        for v in range(H // 16):
                            r = finebuf[fcur, pl.ds(kk * H + v * 16, 16)]
                            a = accb[pl.ds(v * 16, 16)]
                            accb[pl.ds(v * 16, 16)] = jnp.maximum(
                                jnp.where(is_new, neg, a), r)
                        nfl = jnp.where(is_new, nfl + 1, nfl)
                        cur = sgk
                    return (cur, nfl, fcur + 1)

                return lax.cond(impure, fine_fn, pure_fn, (cur, nfl, fcur))

            cur, nfl, _ = lax.fori_loop(
                0, GCH, gbody, (cur, nfl, jnp.int32(0)))
            return (cur, nfl)

        cur, nfl = lax.fori_loop(
            0, NGCHP, chunk_body, (cur0, jnp.int32(0)))

        @pl.when(nfl == 0)
        def _():
            pltpu.sync_copy(accb, fl_sh.at[pl.ds((2 * s) * H, H)])
            istg[pl.ds(0, 16)] = jnp.full((16,), 1, jnp.int32) * cur
            pltpu.sync_copy(istg, flid_sh.at[pl.ds((2 * s) * 16, 16)])
            istg[pl.ds(0, 16)] = jnp.full((16,), -1, jnp.int32)
            pltpu.sync_copy(istg, flid_sh.at[pl.ds((2 * s + 1) * 16, 16)])

        @pl.when(nfl > 0)
        def _():
            pltpu.sync_copy(accb, fl_sh.at[pl.ds((2 * s + 1) * H, H)])
            istg[pl.ds(0, 16)] = jnp.full((16,), 1, jnp.int32) * cur
            pltpu.sync_copy(istg, flid_sh.at[pl.ds((2 * s + 1) * 16, 16)])

        plsc.subcore_barrier()

        @pl.when(s == 0)
        def _():
            pltpu.sync_copy(fl_sh, flbuf)
            pltpu.sync_copy(flid_sh, idl)

            def cb(j, st):
                cur2, b0, b1, b2, b3 = st
                idj = idl[pl.ds(j * 16, 16)][0]
                n0 = flbuf[pl.ds(j * H, 16)]
                n1 = flbuf[pl.ds(j * H + 16, 16)]
                n2 = flbuf[pl.ds(j * H + 32, 16)]
                n3 = flbuf[pl.ds(j * H + 48, 16)]
                skip = idj < 0
                same = idj == cur2

                @pl.when(jnp.logical_and(~skip,
                                         jnp.logical_and(~same, cur2 >= 0)))
                def _():
                    _store_row(stg, b0, b1, b2, b3)
                    pltpu.sync_copy(stg, t_sh.at[pl.ds(cur2 * H, H)])

                ncur = jnp.where(skip, cur2, idj)
                fresh = jnp.logical_and(~skip, ~same)
                nb0 = jnp.where(skip, b0,
                                jnp.where(fresh, n0, jnp.maximum(b0, n0)))
                nb1 = jnp.where(skip, b1,
                                jnp.where(fresh, n1, jnp.maximum(b1, n1)))
                nb2 = jnp.where(skip, b2,
                                jnp.where(fresh, n2, jnp.maximum(b2, n2)))
                nb3 = jnp.where(skip, b3,
                                jnp.where(fresh, n3, jnp.maximum(b3, n3)))
                return (ncur, nb0, nb1, nb2, nb3)

            cur2, b0, b1, b2, b3 = lax.fori_loop(
                0, 2 * NS, cb, (jnp.int32(-1), neg, neg, neg, neg))

            @pl.when(cur2 >= 0)
            def _():
                _store_row(stg, b0, b1, b2, b3)
                pltpu.sync_copy(stg, t_sh.at[pl.ds(cur2 * H, H)])

        plsc.subcore_barrier()

        pltpu.sync_copy(
            t_sh.at[pl.ds(s * ROWS_PER_S * H, ROWS_PER_S * H)],
            out_hbm.at[c, pl.ds(s * ROWS_PER_S * H, ROWS_PER_S * H)])

    return k


def _make_expand(NP, ECH=ECH):
    CHP = NP // NW
    NECHP = CHP // ECH
    mesh = plsc.VectorSubcoreMesh(core_axis_name="c", subcore_axis_name="s",
                                  num_cores=NC, num_subcores=NS)

    @functools.partial(
        pl.kernel,
        out_type=jax.ShapeDtypeStruct((NP, H), jnp.float32),
        mesh=mesh,
        compiler_params=pltpu.CompilerParams(use_tc_tiling_on_sc=False),
        scratch_types=[
            pltpu.VMEM((ECH,), jnp.int32),
            pltpu.VMEM((ECH,), jnp.int32),
            pltpu.VMEM((ECH, H), jnp.float32),
            pltpu.VMEM((ECH, H), jnp.float32),
            pltpu.SemaphoreType.DMA,
            pltpu.SemaphoreType.DMA,
        ],
    )
    def k(t_hbm, idx_hbm, out_hbm, ib0, ib1, rb0, rb1, s0, s1):
        c = lax.axis_index("c")
        s = lax.axis_index("s")
        base = (c * NS + s) * CHP
        ibs = (ib0, ib1)
        rbs = (rb0, rb1)
        sems = (s0, s1)

        pltpu.sync_copy(idx_hbm.at[pl.ds(base, ECH)], ib0)
        gh = pltpu.async_copy(t_hbm.at[ib0], rb0, s0)
        for i in range(NECHP):
            p = i % 2
            q = (i + 1) % 2
            if i + 1 < NECHP:
                pltpu.sync_copy(
                    idx_hbm.at[pl.ds(base + (i + 1) * ECH, ECH)], ibs[q])
                gh_next = pltpu.async_copy(t_hbm.at[ibs[q]], rbs[q], sems[q])
            gh.wait()
            pltpu.sync_copy(rbs[p], out_hbm.at[pl.ds(base + i * ECH, ECH), :])
            if i + 1 < NECHP:
                gh = gh_next

    return k


def kernel(pts, idx, n_idx, W_pos, b_pos, W1, b1, W2, b2, W3, b3, W4, b4,
           W_out, b_out):
    idx = idx.astype(jnp.int32)
    bp = b_pos.reshape(1, H)
    b1r = b1.reshape(1, H)
    b2r = b2.reshape(1, H)
    b3r = b3.reshape(1, H)
    b4r = b4.reshape(1, H)
    bor = b_out.reshape(1, H)

    NSP = 5
    NPH = NPTS // NSP
    NGH = NPH // G
    pool_h = _make_pool(NPH, GCH=100, FBR=112)
    exp_h = _make_expand(NPH, ECH=800)

    parts = []
    for q in range(NSP):
        idxh = idx[q * NPH:(q + 1) * NPH]
        igfh = idxh[0::G]
        iglh = idxh[G - 1::G]
        idx16h = jnp.pad(idxh.reshape(NGH, G), ((0, 0), (0, 16 - G)))
        parts.append((idxh, igfh, iglh, idx16h))

    def pool(hh, gg, hv):
        _, igfh, iglh, idx16h = hv
        return pool_h(hh.reshape(NGH, G * H), gg.reshape(-1), igfh, iglh,
                      idx16h).reshape(NC, NSEGP, H)

    def expand(tt, hv):
        return exp_h(tt, hv[0])

    hs = []
    for q in range(NSP):
        hs.append(_layer1(pts[q * NPH:(q + 1) * NPH], W_pos, bp, W1, b1r))

    for (Wk, bk) in ((W2, b2r), (W3, b3r), (W4, b4r)):
        Ts = [pool(hq, gq, parts[q]) for q, (hq, gq) in enumerate(hs)]
        t = _table5(Ts, Wk[H:], bk)
        hs = [_mid(hs[q][0], expand(t, parts[q]), Wk[:H])
              for q in range(NSP)]

    Ts = [pool(hq, gq, parts[q]) for q, (hq, gq) in enumerate(hs)]
    out = _table5(Ts, W_out, bor)
    return out[:NSEG]
